# Initial kernel scaffold; baseline (speedup 1.0000x reference)
import jax
import jax.numpy as jnp
from jax.experimental import pallas as pl


def kernel(p, x, o, n, Wq, Wk, Wv, Wp1, bp1, Wp2, bp2, Wo, bo):
    raise NotImplementedError("write your pallas kernel here")


# speedup vs baseline: 8.3183x; 8.3183x over previous
import functools

import jax
import jax.numpy as jnp
import numpy as np
from jax import lax
from jax.experimental import pallas as pl
from jax.experimental.pallas import tpu as pltpu
from jax.experimental.pallas import tpu_sc as plsc

N = 10000
D = 128
K = 16
H = 4
HID = 128
HD = HID // H

NPAD = 10112
RB = 200
RT = 200
RK = RT * K

TD = 256

_NW = 32
_BPW = (N * K) // _NW
_CH = 40
_NCH = _BPW // _CH


def _knn_body(q_ref, pc_ref, out_ref):
    q = q_ref[...]
    pc = pc_ref[...]
    q2 = jnp.sum(q * q, axis=1, keepdims=True)
    p2 = jnp.sum(pc * pc, axis=0, keepdims=True)
    dot = lax.dot_general(q, pc, (((1,), (0,)), ((), ())),
                          preferred_element_type=jnp.float32)
    d = q2 + p2 - 2.0 * dot

    G = NPAD // 128
    gmin = d[:, 0:128]
    for c in range(1, G):
        gmin = jnp.minimum(gmin, d[:, c * 128:(c + 1) * 128])

    liota = lax.broadcasted_iota(jnp.int32, (RB, 128), 1)
    lsel = []
    for _ in range(K):
        m = jnp.min(gmin, axis=1, keepdims=True)
        lj = jnp.min(jnp.where(gmin <= m, liota, 128), axis=1, keepdims=True)
        lsel.append(lj)
        gmin = jnp.where(liota == lj, jnp.float32(3e38), gmin)
    lidx = jnp.concatenate(lsel, axis=1)

    cands, cols = [], []
    for c in range(G):
        cands.append(jnp.take_along_axis(
            d[:, c * 128:(c + 1) * 128], lidx, axis=1))
        cols.append(c * 128 + lidx)
    cand = jnp.concatenate(cands, axis=1)
    col = jnp.concatenate(cols, axis=1)

    out_cols = []
    for _ in range(K):
        m = jnp.min(cand, axis=1, keepdims=True)
        idxj = jnp.min(jnp.where(cand <= m, col, 2 * NPAD),
                       axis=1, keepdims=True)
        out_cols.append(idxj)
        cand = jnp.where(col == idxj, jnp.float32(3e38), cand)
    out_ref[...] = jnp.concatenate(out_cols, axis=1)


def _knn(p, interpret=False):
    q = jnp.pad(p, ((0, 0), (0, 5)))
    pc = jnp.concatenate(
        [p.T, jnp.full((3, NPAD - N), 1e4, jnp.float32)], axis=1)
    pc = jnp.pad(pc, ((0, 5), (0, 0)))
    return pl.pallas_call(
        _knn_body,
        grid=(N // RB,),
        in_specs=[
            pl.BlockSpec((RB, 8), lambda i: (i, 0)),
            pl.BlockSpec((8, NPAD), lambda i: (0, 0)),
        ],
        out_specs=pl.BlockSpec((RB, K), lambda i: (i, 0)),
        out_shape=jax.ShapeDtypeStruct((N, K), jnp.int32),
        interpret=interpret,
    )(q, pc)


def _sc_gather(table, idx3):
    mesh = plsc.VectorSubcoreMesh(core_axis_name="c", subcore_axis_name="s")

    @functools.partial(
        pl.kernel,
        mesh=mesh,
        out_type=jax.ShapeDtypeStruct((N * K, TD), jnp.float32),
        scratch_types=[
            pltpu.VMEM((_NCH, _CH), jnp.int32),
            pltpu.VMEM((2, _CH, TD), jnp.float32),
            pltpu.SemaphoreType.DMA((2,)),
            pltpu.SemaphoreType.DMA((2,)),
        ],
    )
    def gk(tab_hbm, idx_hbm, g_hbm, idx_v, rows_v, gsem, wsem):
        wid = lax.axis_index("s") * 2 + lax.axis_index("c")
        pltpu.sync_copy(idx_hbm.at[wid], idx_v)
        base = wid * _BPW

        def gcopy(c, buf):
            return pltpu.make_async_copy(
                tab_hbm.at[idx_v.at[c]], rows_v.at[buf], gsem.at[buf])

        def wcopy(c, buf):
            return pltpu.make_async_copy(
                rows_v.at[buf], g_hbm.at[pl.ds(base + c * _CH, _CH)],
                wsem.at[buf])

        gcopy(0, 0).start()

        def body(c, carry):
            buf = c & 1

            @pl.when(c >= 1)
            def _():
                wcopy(c - 1, 1 - buf).wait()

            @pl.when(c + 1 < _NCH)
            def _():
                gcopy(c + 1, 1 - buf).start()

            gcopy(c, buf).wait()
            wcopy(c, buf).start()
            return carry

        lax.fori_loop(0, _NCH, body, 0)
        wcopy(_NCH - 1, (_NCH - 1) & 1).wait()

    return gk(table, idx3)


PPB = 1250


def _ppf_body(geo_ref, out_ref):
    g = geo_ref[...]
    cpx, cpy, cpz = g[0], g[1], g[2]
    cnx, cny, cnz = g[3], g[4], g[5]
    pix, piy, piz = g[6], g[7], g[8]
    nix, niy, niz = g[9], g[10], g[11]

    dvx, dvy, dvz = cpx - pix, cpy - piy, cpz - piz

    def ang(ux, uy, uz, vx, vy, vz):
        cx = uy * vz - uz * vy
        cy = uz * vx - ux * vz
        cz = ux * vy - uy * vx
        c = jnp.sqrt(cx * cx + cy * cy + cz * cz + 1e-12)
        dt = ux * vx + uy * vy + uz * vz
        return jnp.arctan2(c, dt)

    out_ref[0] = ang(nix, niy, niz, dvx, dvy, dvz)
    out_ref[1] = ang(cnx, cny, cnz, dvx, dvy, dvz)
    out_ref[2] = ang(nix, niy, niz, cnx, cny, cnz)
    out_ref[3] = jnp.sqrt(dvx * dvx + dvy * dvy + dvz * dvz + 1e-12)


def _ppf_planes(geo, interpret=False):
    rows = (N * K) // 128
    return pl.pallas_call(
        _ppf_body,
        grid=(rows // PPB,),
        in_specs=[pl.BlockSpec((12, PPB, 128), lambda i: (0, i, 0))],
        out_specs=pl.BlockSpec((4, PPB, 128), lambda i: (0, i, 0)),
        out_shape=jax.ShapeDtypeStruct((4, rows, 128), jnp.float32),
        interpret=interpret,
    )(geo)


def _tf_body(g_ref, xq_ref, ppf_ref, wq_ref, wk_ref, wv_ref,
             wp1_ref, bp1_ref, wp2_ref, bp2_ref, wo_ref, bo_ref, out_ref):
    xg = g_ref[...]

    pe = jax.nn.relu(
        jnp.dot(ppf_ref[...], wp1_ref[...], preferred_element_type=jnp.float32)
        + bp1_ref[...])
    pe = jnp.dot(pe, wp2_ref[...],
                 preferred_element_type=jnp.float32) + bp2_ref[...]

    q = jnp.dot(xq_ref[...], wq_ref[...],
                preferred_element_type=jnp.float32)
    kk = jnp.dot(xg, wk_ref[...], preferred_element_type=jnp.float32) + pe
    vv = jnp.dot(xg, wv_ref[...], preferred_element_type=jnp.float32) + pe

    di = lax.broadcasted_iota(jnp.int32, (HID, H), 0) // HD
    hi = lax.broadcasted_iota(jnp.int32, (HID, H), 1)
    hm = (di == hi).astype(jnp.float32)
    dit = lax.broadcasted_iota(jnp.int32, (H, HID), 1) // HD
    hit = lax.broadcasted_iota(jnp.int32, (H, HID), 0)
    hmT = (dit == hit).astype(jnp.float32)

    qb = jnp.broadcast_to(q[:, None, :], (RT, K, HID)).reshape(RK, HID)
    logits = jnp.dot(qb * kk, hm,
                     preferred_element_type=jnp.float32)
    logits = logits * jnp.float32(1.0 / np.sqrt(HD))

    l3 = logits.reshape(RT, K, H)
    mx = jnp.max(l3, axis=1, keepdims=True)
    e = jnp.exp(l3 - mx)
    attn = (e / jnp.sum(e, axis=1, keepdims=True)).reshape(RK, H)

    a128 = jnp.dot(attn, hmT, preferred_element_type=jnp.float32)
    out = jnp.sum((a128 * vv).reshape(RT, K, HID), axis=1)
    out_ref[...] = jnp.dot(out, wo_ref[...],
                           preferred_element_type=jnp.float32) + bo_ref[...]


def _transformer(g, x, ppf8, Wq, Wk, Wv, Wp1p, bp1, Wp2, bp2, Wo, bo,
                 interpret=False):
    full = lambda s: pl.BlockSpec(s, lambda i: tuple(0 for _ in s))
    return pl.pallas_call(
        _tf_body,
        grid=(N // RT,),
        in_specs=[
            pl.BlockSpec((RK, D), lambda i: (i, 0)),
            pl.BlockSpec((RT, D), lambda i: (i, 0)),
            pl.BlockSpec((RK, 8), lambda i: (i, 0)),
            full((D, HID)), full((D, HID)), full((D, HID)),
            full((8, HID)), full((1, HID)),
            full((HID, HID)), full((1, HID)),
            full((HID, D)), full((1, D)),
        ],
        out_specs=pl.BlockSpec((RT, D), lambda i: (i, 0)),
        out_shape=jax.ShapeDtypeStruct((N, D), jnp.float32),
        interpret=interpret,
    )(g, x, ppf8, Wq, Wk, Wv, Wp1p, bp1, Wp2, bp2, Wo, bo)


def kernel(p, x, o, n, Wq, Wk, Wv, Wp1, bp1, Wp2, bp2, Wo, bo):
    group_idx = _knn(p)

    table = jnp.concatenate(
        [x, p, n, jnp.zeros((N, TD - D - 6), jnp.float32)], axis=1)
    idx3 = group_idx.reshape(_NW, _NCH, _CH)
    g = _sc_gather(table, idx3)

    NK = N * K
    pn_rep = jnp.repeat(jnp.concatenate([p, n], axis=1), K, axis=0)
    geo = jnp.concatenate([g[:, D:D + 6], pn_rep], axis=1)
    geo = geo.T.reshape(12, NK // 128, 128)
    ppfp = _ppf_planes(geo)
    ppf8 = jnp.pad(ppfp.reshape(4, NK).T, ((0, 0), (0, 4)))

    Wp1p = jnp.pad(Wp1, ((0, 4), (0, 0)))
    x_out = _transformer(
        g, x, ppf8, Wq, Wk, Wv, Wp1p,
        bp1.reshape(1, HID), Wp2, bp2.reshape(1, HID), Wo, bo.reshape(1, D))

    idx = jnp.arange(N, dtype=jnp.int32)
    return (p, x_out, o, n, idx)
